# baseline (speedup 1.0000x reference)
import functools

import jax
import jax.numpy as jnp
from jax import lax
from jax.experimental import pallas as pl
from jax.experimental.pallas import tpu as pltpu
from jax.experimental.pallas import tpu_sc as plsc

EMBED = 16
BATCH = 16384
NC = 2
NS = 16
NW = NC * NS
PER_W = BATCH // NW
CHUNK = 128
NCHUNK = PER_W // CHUNK
FLAT = PER_W * EMBED


def _sc_gather_dot(idx_u3, idx_m3, uemb_t, memb_t, ub_flat, mb_flat):
    mesh = plsc.VectorSubcoreMesh(core_axis_name="c", subcore_axis_name="s")

    @functools.partial(
        pl.kernel,
        mesh=mesh,
        compiler_params=pltpu.CompilerParams(use_tc_tiling_on_sc=False),
        out_type=(
            jax.ShapeDtypeStruct((NW, EMBED), jnp.float32),
            jax.ShapeDtypeStruct((BATCH,), jnp.float32),
            jax.ShapeDtypeStruct((BATCH,), jnp.float32),
        ),
        scratch_types=[
            pltpu.VMEM((NCHUNK, CHUNK), jnp.int32),
            pltpu.VMEM((NCHUNK, CHUNK), jnp.int32),
            pltpu.VMEM((FLAT,), jnp.float32),
            pltpu.VMEM((FLAT,), jnp.float32),
            pltpu.VMEM((PER_W,), jnp.float32),
            pltpu.VMEM((PER_W,), jnp.float32),
            pltpu.VMEM((EMBED,), jnp.float32),
            pltpu.SemaphoreType.DMA,
            pltpu.SemaphoreType.DMA,
        ],
    )
    def k(idx_u_hbm, idx_m_hbm, uemb_hbm, memb_hbm, ub_hbm, mb_hbm,
          partials_hbm, ubg_hbm, mbg_hbm,
          idxu_v, idxm_v, u_v, m_v, bu_v, bm_v, acc_v,
          sem_emb, sem_bias):
        wid = lax.axis_index("s") * NC + lax.axis_index("c")
        base = wid * PER_W
        pltpu.sync_copy(idx_u_hbm.at[wid], idxu_v)
        pltpu.sync_copy(idx_m_hbm.at[wid], idxm_v)
        emb_cps = []
        bias_cps = []
        for t in range(NCHUNK):
            sl = pl.ds(t * CHUNK, CHUNK)
            for e in range(EMBED):
                dsl = pl.ds(e * PER_W + t * CHUNK, CHUNK)
                emb_cps.append(pltpu.async_copy(
                    uemb_hbm.at[e].at[idxu_v.at[t]], u_v.at[dsl], sem_emb))
                emb_cps.append(pltpu.async_copy(
                    memb_hbm.at[e].at[idxm_v.at[t]], m_v.at[dsl], sem_emb))
            bias_cps.append(pltpu.async_copy(
                ub_hbm.at[idxu_v.at[t]], bu_v.at[sl], sem_bias))
            bias_cps.append(pltpu.async_copy(
                mb_hbm.at[idxm_v.at[t]], bm_v.at[sl], sem_bias))
        for cp in emb_cps:
            cp.wait()

        def body(i, acc):
            return acc + u_v[pl.ds(i * 16, 16)] * m_v[pl.ds(i * 16, 16)]

        acc = lax.fori_loop(0, FLAT // 16, body,
                            jnp.zeros((EMBED,), jnp.float32), unroll=8)
        acc_v[...] = acc
        pltpu.sync_copy(acc_v, partials_hbm.at[wid])
        for cp in bias_cps:
            cp.wait()
        pltpu.sync_copy(bu_v, ubg_hbm.at[pl.ds(base, PER_W)])
        pltpu.sync_copy(bm_v, mbg_hbm.at[pl.ds(base, PER_W)])

    return k(idx_u3, idx_m3, uemb_t, memb_t, ub_flat, mb_flat)


def _tc_finish(partials, ubg, mbg):
    def body(p_ref, ub_ref, mb_ref, o_ref):
        s = jnp.sum(p_ref[...])
        x = s + ub_ref[...] + mb_ref[...]
        o_ref[...] = 0.5 + 4.5 * jax.nn.sigmoid(x)

    return pl.pallas_call(
        body,
        out_shape=jax.ShapeDtypeStruct((128, 128), jnp.float32),
    )(partials, ubg, mbg)


def kernel(inputs, user_emb, user_bias_tbl, movie_emb, movie_bias_tbl):
    idx = inputs.astype(jnp.int32)
    idx_u3 = idx[:, 0].reshape(NW, NCHUNK, CHUNK)
    idx_m3 = idx[:, 1].reshape(NW, NCHUNK, CHUNK)
    partials, ubg, mbg = _sc_gather_dot(
        idx_u3, idx_m3, user_emb.T, movie_emb.T,
        user_bias_tbl.reshape(-1), movie_bias_tbl.reshape(-1))
    out = _tc_finish(partials, ubg.reshape(128, 128), mbg.reshape(128, 128))
    return out.reshape(BATCH, 1)


# speedup vs baseline: 3.4840x; 3.4840x over previous
import functools

import jax
import jax.numpy as jnp
from jax import lax
from jax.experimental import pallas as pl
from jax.experimental.pallas import tpu as pltpu
from jax.experimental.pallas import tpu_sc as plsc

EMBED = 16
BATCH = 16384
NROWS = 1000000
NC = 2
NS = 16
NW = NC * NS
PER_W = BATCH // NW
CHUNK = 128
NCHUNK = PER_W // CHUNK
FLAT = PER_W * EMBED


def _sc_gather_dot(idx_u3, idx_m3, u_cols, m_cols, ub_flat, mb_flat):
    mesh = plsc.VectorSubcoreMesh(core_axis_name="c", subcore_axis_name="s")

    @functools.partial(
        pl.kernel,
        mesh=mesh,
        compiler_params=pltpu.CompilerParams(use_tc_tiling_on_sc=False),
        out_type=(
            jax.ShapeDtypeStruct((NW, EMBED), jnp.float32),
            jax.ShapeDtypeStruct((BATCH,), jnp.float32),
            jax.ShapeDtypeStruct((BATCH,), jnp.float32),
        ),
        scratch_types=[
            pltpu.VMEM((NCHUNK, CHUNK), jnp.int32),
            pltpu.VMEM((NCHUNK, CHUNK), jnp.int32),
            pltpu.VMEM((FLAT,), jnp.float32),
            pltpu.VMEM((FLAT,), jnp.float32),
            pltpu.VMEM((PER_W,), jnp.float32),
            pltpu.VMEM((PER_W,), jnp.float32),
            pltpu.VMEM((EMBED,), jnp.float32),
            pltpu.SemaphoreType.DMA,
            pltpu.SemaphoreType.DMA,
        ],
    )
    def k(idx_u_hbm, idx_m_hbm, *rest):
        ucol_refs = rest[:EMBED]
        mcol_refs = rest[EMBED:2 * EMBED]
        ub_hbm, mb_hbm = rest[2 * EMBED], rest[2 * EMBED + 1]
        partials_hbm, ubg_hbm, mbg_hbm = rest[2 * EMBED + 2:2 * EMBED + 5]
        (idxu_v, idxm_v, u_v, m_v, bu_v, bm_v, acc_v,
         sem_emb, sem_bias) = rest[2 * EMBED + 5:]
        wid = lax.axis_index("s") * NC + lax.axis_index("c")
        base = wid * PER_W
        pltpu.sync_copy(idx_u_hbm.at[wid], idxu_v)
        pltpu.sync_copy(idx_m_hbm.at[wid], idxm_v)
        emb_cps = []
        bias_cps = []
        for t in range(NCHUNK):
            sl = pl.ds(t * CHUNK, CHUNK)
            for e in range(EMBED):
                dsl = pl.ds(e * PER_W + t * CHUNK, CHUNK)
                emb_cps.append(pltpu.async_copy(
                    ucol_refs[e].at[idxu_v.at[t]], u_v.at[dsl], sem_emb))
                emb_cps.append(pltpu.async_copy(
                    mcol_refs[e].at[idxm_v.at[t]], m_v.at[dsl], sem_emb))
            bias_cps.append(pltpu.async_copy(
                ub_hbm.at[idxu_v.at[t]], bu_v.at[sl], sem_bias))
            bias_cps.append(pltpu.async_copy(
                mb_hbm.at[idxm_v.at[t]], bm_v.at[sl], sem_bias))
        for cp in emb_cps:
            cp.wait()

        def body(i, acc):
            return acc + u_v[pl.ds(i * 16, 16)] * m_v[pl.ds(i * 16, 16)]

        acc = lax.fori_loop(0, FLAT // 16, body,
                            jnp.zeros((EMBED,), jnp.float32), unroll=8)
        acc_v[...] = acc
        pltpu.sync_copy(acc_v, partials_hbm.at[wid])
        for cp in bias_cps:
            cp.wait()
        pltpu.sync_copy(bu_v, ubg_hbm.at[pl.ds(base, PER_W)])
        pltpu.sync_copy(bm_v, mbg_hbm.at[pl.ds(base, PER_W)])

    return k(idx_u3, idx_m3, *u_cols, *m_cols, ub_flat, mb_flat)


def _tc_finish(partials, ubg, mbg):
    def body(p_ref, ub_ref, mb_ref, o_ref):
        s = jnp.sum(p_ref[...])
        x = s + ub_ref[...] + mb_ref[...]
        o_ref[...] = 0.5 + 4.5 * jax.nn.sigmoid(x)

    return pl.pallas_call(
        body,
        out_shape=jax.ShapeDtypeStruct((128, 128), jnp.float32),
    )(partials, ubg, mbg)


def kernel(inputs, user_emb, user_bias_tbl, movie_emb, movie_bias_tbl):
    idx = inputs.astype(jnp.int32)
    idx_u3 = idx[:, 0].reshape(NW, NCHUNK, CHUNK)
    idx_m3 = idx[:, 1].reshape(NW, NCHUNK, CHUNK)
    u_cols = [user_emb[:, e] for e in range(EMBED)]
    m_cols = [movie_emb[:, e] for e in range(EMBED)]
    partials, ubg, mbg = _sc_gather_dot(
        idx_u3, idx_m3, u_cols, m_cols,
        user_bias_tbl.reshape(-1), movie_bias_tbl.reshape(-1))
    out = _tc_finish(partials, ubg.reshape(128, 128), mbg.reshape(128, 128))
    return out.reshape(BATCH, 1)
